# baseline (speedup 1.0000x reference)
import jax
import jax.numpy as jnp
from jax.experimental import pallas as pl
from jax.experimental.pallas import tpu as pltpu

NUM_EXPERTS = 64
TOP_K = 8
D_MODEL = 4096
TOKENS = 16384

T_BLK = 1024
N_CHUNKS = 8
D_CHUNK = D_MODEL // N_CHUNKS


def _router_block(*refs):
    x_refs = refs[:N_CHUNKS]
    wt_ref, b_ref, w_out, i_out, psum_out, zsum_out = refs[N_CHUNKS:]
    logits = jnp.dot(
        x_refs[0][...], wt_ref[0:D_CHUNK, :],
        preferred_element_type=jnp.float32,
    )
    for j in range(1, N_CHUNKS):
        logits += jnp.dot(
            x_refs[j][...], wt_ref[j * D_CHUNK:(j + 1) * D_CHUNK, :],
            preferred_element_type=jnp.float32,
        )

    lt = logits.T + b_ref[...]
    zsum_out[0, 0, :] = jnp.sum(lt * lt, axis=1)

    m = jnp.max(lt, axis=0, keepdims=True)
    e = jnp.exp(lt - m)
    s = jnp.sum(e, axis=0, keepdims=True)
    probs = e / s

    psum_out[0, 0, :] = jnp.sum(probs, axis=1)

    sub = jax.lax.broadcasted_iota(jnp.int32, probs.shape, 0)
    vals = probs
    ws = []
    idxs = []
    for _ in range(TOP_K):
        mk = jnp.max(vals, axis=0, keepdims=True)
        is_mk = vals >= mk
        idx = jnp.min(
            jnp.where(is_mk, sub, NUM_EXPERTS), axis=0, keepdims=True
        )
        ws.append(mk)
        idxs.append(idx)
        vals = jnp.where(sub == idx, -1.0, vals)

    w_cat = jnp.concatenate(ws, axis=0)
    wsum = jnp.sum(w_cat, axis=0, keepdims=True)
    w_out[...] = (w_cat / (wsum + 1e-8)).T
    i_out[...] = jnp.concatenate(idxs, axis=0).T


@jax.jit
def kernel(x, W, expert_bias):
    grid = TOKENS // T_BLK
    w_t = W.T
    bias = expert_bias.reshape(NUM_EXPERTS, 1)

    w_out, i_out, psum, zsum = pl.pallas_call(
        _router_block,
        grid=(grid,),
        in_specs=[
            pl.BlockSpec((T_BLK, D_CHUNK), lambda i, j=j: (i, j))
            for j in range(N_CHUNKS)
        ] + [
            pl.BlockSpec((D_MODEL, NUM_EXPERTS), lambda i: (0, 0)),
            pl.BlockSpec((NUM_EXPERTS, 1), lambda i: (0, 0)),
        ],
        out_specs=[
            pl.BlockSpec((T_BLK, TOP_K), lambda i: (i, 0)),
            pl.BlockSpec((T_BLK, TOP_K), lambda i: (i, 0)),
            pl.BlockSpec((1, 1, NUM_EXPERTS), lambda i: (i, 0, 0)),
            pl.BlockSpec((1, 1, NUM_EXPERTS), lambda i: (i, 0, 0)),
        ],
        out_shape=[
            jax.ShapeDtypeStruct((TOKENS, TOP_K), jnp.float32),
            jax.ShapeDtypeStruct((TOKENS, TOP_K), jnp.int32),
            jax.ShapeDtypeStruct((grid, 1, NUM_EXPERTS), jnp.float32),
            jax.ShapeDtypeStruct((grid, 1, NUM_EXPERTS), jnp.float32),
        ],
        compiler_params=pltpu.CompilerParams(
            dimension_semantics=("parallel",),
        ),
    )(*([x] * N_CHUNKS), w_t, bias)

    tokens_per_expert = jnp.sum(psum, axis=(0, 1)) / TOKENS
    uniform = 1.0 / NUM_EXPERTS
    load_balance_loss = (
        jnp.sum((tokens_per_expert - uniform) ** 2) * NUM_EXPERTS
    )
    router_z_loss = jnp.sum(zsum) / (TOKENS * NUM_EXPERTS) * 0.001
    return (w_out, i_out, load_balance_loss, router_z_loss)


# speedup vs baseline: 1.0004x; 1.0004x over previous
import jax
import jax.numpy as jnp
from jax.experimental import pallas as pl
from jax.experimental.pallas import tpu as pltpu

NUM_EXPERTS = 64
TOP_K = 8
D_MODEL = 4096
TOKENS = 16384

T_BLK = 1024
N_CHUNKS = 8
T_CHUNK = T_BLK // N_CHUNKS


def _router_block(*refs):
    x_refs = refs[:N_CHUNKS]
    wt_ref, b_ref, w_out, i_out, psum_out, zsum_out = refs[N_CHUNKS:]
    logits = jnp.concatenate(
        [
            jnp.dot(xr[...], wt_ref[...], preferred_element_type=jnp.float32)
            for xr in x_refs
        ],
        axis=0,
    )

    lt = logits.T + b_ref[...]
    zsum_out[0, 0, :] = jnp.sum(lt * lt, axis=1)

    m = jnp.max(lt, axis=0, keepdims=True)
    e = jnp.exp(lt - m)
    s = jnp.sum(e, axis=0, keepdims=True)
    probs = e / s

    psum_out[0, 0, :] = jnp.sum(probs, axis=1)

    sub = jax.lax.broadcasted_iota(jnp.int32, probs.shape, 0)
    vals = probs
    ws = []
    idxs = []
    for _ in range(TOP_K):
        mk = jnp.max(vals, axis=0, keepdims=True)
        is_mk = vals >= mk
        idx = jnp.min(
            jnp.where(is_mk, sub, NUM_EXPERTS), axis=0, keepdims=True
        )
        ws.append(mk)
        idxs.append(idx)
        vals = jnp.where(sub == idx, -1.0, vals)

    w_cat = jnp.concatenate(ws, axis=0)
    wsum = jnp.sum(w_cat, axis=0, keepdims=True)
    w_out[...] = (w_cat / (wsum + 1e-8)).T
    i_out[...] = jnp.concatenate(idxs, axis=0).T


@jax.jit
def kernel(x, W, expert_bias):
    grid = TOKENS // T_BLK
    w_t = W.T
    bias = expert_bias.reshape(NUM_EXPERTS, 1)

    w_out, i_out, psum, zsum = pl.pallas_call(
        _router_block,
        grid=(grid,),
        in_specs=[
            pl.BlockSpec((T_CHUNK, D_MODEL), lambda i, j=j: (i * N_CHUNKS + j, 0))
            for j in range(N_CHUNKS)
        ] + [
            pl.BlockSpec((D_MODEL, NUM_EXPERTS), lambda i: (0, 0)),
            pl.BlockSpec((NUM_EXPERTS, 1), lambda i: (0, 0)),
        ],
        out_specs=[
            pl.BlockSpec((T_BLK, TOP_K), lambda i: (i, 0)),
            pl.BlockSpec((T_BLK, TOP_K), lambda i: (i, 0)),
            pl.BlockSpec((1, 1, NUM_EXPERTS), lambda i: (i, 0, 0)),
            pl.BlockSpec((1, 1, NUM_EXPERTS), lambda i: (i, 0, 0)),
        ],
        out_shape=[
            jax.ShapeDtypeStruct((TOKENS, TOP_K), jnp.float32),
            jax.ShapeDtypeStruct((TOKENS, TOP_K), jnp.int32),
            jax.ShapeDtypeStruct((grid, 1, NUM_EXPERTS), jnp.float32),
            jax.ShapeDtypeStruct((grid, 1, NUM_EXPERTS), jnp.float32),
        ],
        compiler_params=pltpu.CompilerParams(
            dimension_semantics=("parallel",),
        ),
    )(*([x] * N_CHUNKS), w_t, bias)

    tokens_per_expert = jnp.sum(psum, axis=(0, 1)) / TOKENS
    uniform = 1.0 / NUM_EXPERTS
    load_balance_loss = (
        jnp.sum((tokens_per_expert - uniform) ** 2) * NUM_EXPERTS
    )
    router_z_loss = jnp.sum(zsum) / (TOKENS * NUM_EXPERTS) * 0.001
    return (w_out, i_out, load_balance_loss, router_z_loss)
